# Initial kernel scaffold; baseline (speedup 1.0000x reference)
import jax
import jax.numpy as jnp
from jax.experimental import pallas as pl


def kernel(x, edge_index, W1, b1, W2, b2, Wl, bl):
    raise NotImplementedError("write your pallas kernel here")


# speedup vs baseline: 13.0299x; 13.0299x over previous
import functools

import jax
import jax.numpy as jnp
from jax import lax
from jax.experimental import pallas as pl
from jax.experimental.pallas import tpu as pltpu
from jax.experimental.pallas import tpu_sc as plsc

N, E, D = 10000, 320000, 128
NCLS, NCLS_PAD = 1250, 1280
NC, NS = 2, 16
NW = NC * NS
EPW = E // NW
CH = 128
NCHUNK = EPW // CH
REM = EPW - NCHUNK * CH
RPS = 640
N_PAD = NS * RPS
RQ = RPS // CH
DEGW = 128


def _mesh():
    return plsc.VectorSubcoreMesh(
        core_axis_name="c", subcore_axis_name="s",
        num_cores=NC, num_subcores=NS)


def _fill_wide(ref, n, val):
    vec = jnp.zeros((16,), jnp.float32) + val

    def body(i, _):
        for j in range(DEGW // 16):
            ref[i, pl.ds(j * 16, 16)] = vec
        return 0
    lax.fori_loop(0, n, body, 0)


def _mkidx(idx_v, base):
    def body(j, _):
        idx_v[pl.ds(j * 16, 16)] = lax.iota(jnp.int32, 16) + base + j * 16
        return 0
    lax.fori_loop(0, CH // 16, body, 0)


def _sc_deg_body(dst_hbm, out_hbm, idx_v, idxr_v, ones_v, stage_v, hist_sh):
    c = lax.axis_index("c")
    s = lax.axis_index("s")
    wid = c * NS + s
    base = s * RPS

    _fill_wide(stage_v, CH, 0.0)
    for q in range(RQ):
        _mkidx(idx_v, base + q * CH)
        pltpu.sync_copy(stage_v, hist_sh.at[idx_v])
    _fill_wide(ones_v, CH, 1.0)
    plsc.subcore_barrier()

    ebase = wid * EPW

    def chunk(i, _):
        pltpu.sync_copy(dst_hbm.at[pl.ds(ebase + i * CH, CH)], idx_v)
        pltpu.sync_copy(ones_v, hist_sh.at[idx_v], add=True)
        return 0
    lax.fori_loop(0, NCHUNK, chunk, 0)

    pltpu.sync_copy(dst_hbm.at[pl.ds(ebase + NCHUNK * CH, REM)], idxr_v)
    pltpu.sync_copy(ones_v.at[pl.ds(0, REM)], hist_sh.at[idxr_v], add=True)

    plsc.subcore_barrier()

    for q in range(RQ):
        _mkidx(idx_v, base + q * CH)
        pltpu.sync_copy(hist_sh.at[idx_v], stage_v)
        pltpu.sync_copy(
            stage_v,
            out_hbm.at[pl.ds(pl.multiple_of(c * N_PAD + base + q * CH, 8),
                             CH)])


@functools.cache
def _sc_deg():
    return pl.kernel(
        _sc_deg_body,
        out_type=jax.ShapeDtypeStruct((NC * N_PAD, DEGW), jnp.float32),
        mesh=_mesh(),
        scratch_types=[
            pltpu.VMEM((CH,), jnp.int32),
            pltpu.VMEM((REM,), jnp.int32),
            pltpu.VMEM((CH, DEGW), jnp.float32),
            pltpu.VMEM((CH, DEGW), jnp.float32),
            pltpu.VMEM_SHARED((N_PAD, DEGW), jnp.float32),
        ],
        name="sc_degree",
    )


def _sc_agg_body(hs_hbm, src_hbm, dst_hbm, out_hbm,
                 si_v, di_v, sir_v, dir_v, rows_v, acc_sh, sem):
    c = lax.axis_index("c")
    s = lax.axis_index("s")
    wid = c * NS + s
    base = s * RPS

    for q in range(RQ):
        _mkidx(si_v, base + q * CH)
        pltpu.sync_copy(
            hs_hbm.at[pl.ds(pl.multiple_of(base + q * CH, 8), CH)], rows_v)
        pltpu.sync_copy(rows_v, acc_sh.at[si_v])
    plsc.subcore_barrier()

    ebase = wid * EPW

    def chunk(i, _):
        off = ebase + i * CH
        pltpu.sync_copy(src_hbm.at[pl.ds(off, CH)], si_v)
        pltpu.sync_copy(dst_hbm.at[pl.ds(off, CH)], di_v)
        pltpu.async_copy(hs_hbm.at[si_v], rows_v, sem).wait()
        pltpu.sync_copy(rows_v, acc_sh.at[di_v], add=True)
        return 0
    lax.fori_loop(0, NCHUNK, chunk, 0)

    off = ebase + NCHUNK * CH
    pltpu.sync_copy(src_hbm.at[pl.ds(off, REM)], sir_v)
    pltpu.sync_copy(dst_hbm.at[pl.ds(off, REM)], dir_v)
    pltpu.async_copy(hs_hbm.at[sir_v], rows_v.at[pl.ds(0, REM)], sem).wait()
    pltpu.sync_copy(rows_v.at[pl.ds(0, REM)], acc_sh.at[dir_v], add=True)

    plsc.subcore_barrier()

    for q in range(RQ):
        _mkidx(si_v, base + q * CH)
        pltpu.sync_copy(acc_sh.at[si_v], rows_v)
        pltpu.sync_copy(
            rows_v,
            out_hbm.at[pl.ds(pl.multiple_of(c * N_PAD + base + q * CH, 8),
                             CH)])


@functools.cache
def _sc_agg():
    return pl.kernel(
        _sc_agg_body,
        out_type=jax.ShapeDtypeStruct((NC * N_PAD, D), jnp.float32),
        mesh=_mesh(),
        scratch_types=[
            pltpu.VMEM((CH,), jnp.int32),
            pltpu.VMEM((CH,), jnp.int32),
            pltpu.VMEM((REM,), jnp.int32),
            pltpu.VMEM((REM,), jnp.int32),
            pltpu.VMEM((CH, D), jnp.float32),
            pltpu.VMEM_SHARED((N_PAD, D), jnp.float32),
            pltpu.SemaphoreType.DMA,
        ],
        name="sc_row_agg",
    )


BR = 400
GRID = N // BR


def _prep_body(deg_ref, x_ref, hs_ref, dis_ref):
    degb = deg_ref[...]
    deg = degb[0, :, 0:1] + degb[1, :, 0:1] + 1.0
    dis = lax.rsqrt(deg)
    dis_ref[...] = dis
    hs_ref[...] = x_ref[...] * dis


def _tc_prep(degp, x):
    return pl.pallas_call(
        _prep_body,
        grid=(GRID,),
        in_specs=[
            pl.BlockSpec((NC, BR, DEGW), lambda i: (0, i, 0)),
            pl.BlockSpec((BR, D), lambda i: (i, 0)),
        ],
        out_specs=(
            pl.BlockSpec((BR, D), lambda i: (i, 0)),
            pl.BlockSpec((BR, 1), lambda i: (i, 0)),
        ),
        out_shape=(
            jax.ShapeDtypeStruct((N, D), jnp.float32),
            jax.ShapeDtypeStruct((N, 1), jnp.float32),
        ),
    )(degp, x)


def _layer1_body(acc_ref, hs_ref, dis_ref, w1_ref, b1_ref, out_ref):
    accb = acc_ref[...]
    a = (accb[0] + accb[1] - hs_ref[...]) * dis_ref[...]
    h1 = jnp.dot(a, w1_ref[...], preferred_element_type=jnp.float32)
    h1 = jnp.maximum(h1 + b1_ref[...], 0.0)
    out_ref[...] = h1 * dis_ref[...]


def _tc_layer1(acc1, hs, dis, W1, b1r):
    return pl.pallas_call(
        _layer1_body,
        grid=(GRID,),
        in_specs=[
            pl.BlockSpec((NC, BR, D), lambda i: (0, i, 0)),
            pl.BlockSpec((BR, D), lambda i: (i, 0)),
            pl.BlockSpec((BR, 1), lambda i: (i, 0)),
            pl.BlockSpec((D, D), lambda i: (0, 0)),
            pl.BlockSpec((1, D), lambda i: (0, 0)),
        ],
        out_specs=pl.BlockSpec((BR, D), lambda i: (i, 0)),
        out_shape=jax.ShapeDtypeStruct((N, D), jnp.float32),
    )(acc1, hs, dis, W1, b1r)


def _layer2_body(acc_ref, h1s_ref, dis_ref, w2_ref, b2_ref, wl_ref, bl_ref,
                 out_ref):
    accb = acc_ref[...]
    a = (accb[0] + accb[1] - h1s_ref[...]) * dis_ref[...]
    h2 = jnp.dot(a, w2_ref[...], preferred_element_type=jnp.float32)
    h2 = h2 + b2_ref[...]
    out = jnp.dot(h2, wl_ref[...], preferred_element_type=jnp.float32)
    out_ref[...] = out + bl_ref[...]


def _tc_layer2(acc2, h1s, dis, W2p, b2p, WlTp, blp):
    return pl.pallas_call(
        _layer2_body,
        grid=(GRID,),
        in_specs=[
            pl.BlockSpec((NC, BR, D), lambda i: (0, i, 0)),
            pl.BlockSpec((BR, D), lambda i: (i, 0)),
            pl.BlockSpec((BR, 1), lambda i: (i, 0)),
            pl.BlockSpec((D, NCLS_PAD), lambda i: (0, 0)),
            pl.BlockSpec((1, NCLS_PAD), lambda i: (0, 0)),
            pl.BlockSpec((NCLS_PAD, NCLS_PAD), lambda i: (0, 0)),
            pl.BlockSpec((1, NCLS_PAD), lambda i: (0, 0)),
        ],
        out_specs=pl.BlockSpec((BR, NCLS_PAD), lambda i: (i, 0)),
        out_shape=jax.ShapeDtypeStruct((N, NCLS_PAD), jnp.float32),
    )(acc2, h1s, dis, W2p, b2p, WlTp, blp)


def kernel(x, edge_index, W1, b1, W2, b2, Wl, bl):
    src = edge_index[0]
    dst = edge_index[1]
    npad = N_PAD - N

    degp = _sc_deg()(dst).reshape(NC, N_PAD, DEGW)[:, :N]
    hs, dis = _tc_prep(degp, x)
    hs_pad = jnp.pad(hs, ((0, npad), (0, 0)))
    acc1 = _sc_agg()(hs_pad, src, dst).reshape(NC, N_PAD, D)[:, :N]
    h1s = _tc_layer1(acc1, hs, dis, W1, b1.reshape(1, D))

    h1s_pad = jnp.pad(h1s, ((0, npad), (0, 0)))
    acc2 = _sc_agg()(h1s_pad, src, dst).reshape(NC, N_PAD, D)[:, :N]
    pad = NCLS_PAD - NCLS
    W2p = jnp.pad(W2, ((0, 0), (0, pad)))
    b2p = jnp.pad(b2, (0, pad)).reshape(1, NCLS_PAD)
    WlTp = jnp.pad(Wl.T, ((0, pad), (0, pad)))
    blp = jnp.pad(bl, (0, pad)).reshape(1, NCLS_PAD)
    outp = _tc_layer2(acc2, h1s, dis, W2p, b2p, WlTp, blp)
    return outp[:, :NCLS]


# speedup vs baseline: 17.2642x; 1.3250x over previous
import functools

import jax
import jax.numpy as jnp
from jax import lax
from jax.experimental import pallas as pl
from jax.experimental.pallas import tpu as pltpu
from jax.experimental.pallas import tpu_sc as plsc

N, E, D = 10000, 320000, 128
NCLS = 1250
NC, NS = 2, 16
NW = NC * NS
EPW = E // NW
CH = 128
NCHUNK = EPW // CH
REM = EPW - NCHUNK * CH
RPS = 640
N_PAD = NS * RPS
RQ = RPS // CH
DEGW = 128


def _mesh():
    return plsc.VectorSubcoreMesh(
        core_axis_name="c", subcore_axis_name="s",
        num_cores=NC, num_subcores=NS)


def _fill_wide(ref, n, val):
    vec = jnp.zeros((16,), jnp.float32) + val

    def body(i, _):
        for j in range(DEGW // 16):
            ref[i, pl.ds(j * 16, 16)] = vec
        return 0
    lax.fori_loop(0, n, body, 0)


def _mkidx(idx_v, base):
    def body(j, _):
        idx_v[pl.ds(j * 16, 16)] = lax.iota(jnp.int32, 16) + base + j * 16
        return 0
    lax.fori_loop(0, CH // 16, body, 0)


def _sc_deg_body(dst_hbm, out0_hbm, out1_hbm,
                 idx_v, idxr_v, ones_v, stage_v, hist_sh):
    c = lax.axis_index("c")
    s = lax.axis_index("s")
    wid = c * NS + s
    base = s * RPS

    _fill_wide(stage_v, CH, 0.0)
    for q in range(RQ):
        _mkidx(idx_v, base + q * CH)
        pltpu.sync_copy(stage_v, hist_sh.at[idx_v])
    _fill_wide(ones_v, CH, 1.0)
    plsc.subcore_barrier()

    ebase = wid * EPW

    def chunk(i, _):
        pltpu.sync_copy(dst_hbm.at[pl.ds(ebase + i * CH, CH)], idx_v)
        pltpu.sync_copy(ones_v, hist_sh.at[idx_v], add=True)
        return 0
    lax.fori_loop(0, NCHUNK, chunk, 0)

    pltpu.sync_copy(dst_hbm.at[pl.ds(ebase + NCHUNK * CH, REM)], idxr_v)
    pltpu.sync_copy(ones_v.at[pl.ds(0, REM)], hist_sh.at[idxr_v], add=True)

    plsc.subcore_barrier()

    def wout(out_hbm):
        for q in range(RQ):
            _mkidx(idx_v, base + q * CH)
            pltpu.sync_copy(hist_sh.at[idx_v], stage_v)
            pltpu.sync_copy(
                stage_v,
                out_hbm.at[pl.ds(pl.multiple_of(base + q * CH, 8), CH)])

    @pl.when(c == 0)
    def _():
        wout(out0_hbm)

    @pl.when(c == 1)
    def _():
        wout(out1_hbm)


@functools.cache
def _sc_deg():
    return pl.kernel(
        _sc_deg_body,
        out_type=(jax.ShapeDtypeStruct((N_PAD, DEGW), jnp.float32),
                  jax.ShapeDtypeStruct((N_PAD, DEGW), jnp.float32)),
        mesh=_mesh(),
        scratch_types=[
            pltpu.VMEM((CH,), jnp.int32),
            pltpu.VMEM((REM,), jnp.int32),
            pltpu.VMEM((CH, DEGW), jnp.float32),
            pltpu.VMEM((CH, DEGW), jnp.float32),
            pltpu.VMEM_SHARED((N_PAD, DEGW), jnp.float32),
        ],
        name="sc_degree",
    )


def _sc_agg_body(hs_hbm, src_hbm, dst_hbm, out0_hbm, out1_hbm,
                 si_v, di_v, sir_v, dir_v, rows_v, acc_sh, sem):
    c = lax.axis_index("c")
    s = lax.axis_index("s")
    wid = c * NS + s
    base = s * RPS

    for q in range(RQ):
        _mkidx(si_v, base + q * CH)
        pltpu.sync_copy(
            hs_hbm.at[pl.ds(pl.multiple_of(base + q * CH, 8), CH)], rows_v)
        pltpu.sync_copy(rows_v, acc_sh.at[si_v])
    plsc.subcore_barrier()

    ebase = wid * EPW

    def chunk(i, _):
        off = ebase + i * CH
        pltpu.sync_copy(src_hbm.at[pl.ds(off, CH)], si_v)
        pltpu.sync_copy(dst_hbm.at[pl.ds(off, CH)], di_v)
        pltpu.async_copy(hs_hbm.at[si_v], rows_v, sem).wait()
        pltpu.sync_copy(rows_v, acc_sh.at[di_v], add=True)
        return 0
    lax.fori_loop(0, NCHUNK, chunk, 0)

    off = ebase + NCHUNK * CH
    pltpu.sync_copy(src_hbm.at[pl.ds(off, REM)], sir_v)
    pltpu.sync_copy(dst_hbm.at[pl.ds(off, REM)], dir_v)
    pltpu.async_copy(hs_hbm.at[sir_v], rows_v.at[pl.ds(0, REM)], sem).wait()
    pltpu.sync_copy(rows_v.at[pl.ds(0, REM)], acc_sh.at[dir_v], add=True)

    plsc.subcore_barrier()

    def wout(out_hbm):
        for q in range(RQ):
            _mkidx(si_v, base + q * CH)
            pltpu.sync_copy(acc_sh.at[si_v], rows_v)
            pltpu.sync_copy(
                rows_v,
                out_hbm.at[pl.ds(pl.multiple_of(base + q * CH, 8), CH)])

    @pl.when(c == 0)
    def _():
        wout(out0_hbm)

    @pl.when(c == 1)
    def _():
        wout(out1_hbm)


@functools.cache
def _sc_agg():
    return pl.kernel(
        _sc_agg_body,
        out_type=(jax.ShapeDtypeStruct((N_PAD, D), jnp.float32),
                  jax.ShapeDtypeStruct((N_PAD, D), jnp.float32)),
        mesh=_mesh(),
        scratch_types=[
            pltpu.VMEM((CH,), jnp.int32),
            pltpu.VMEM((CH,), jnp.int32),
            pltpu.VMEM((REM,), jnp.int32),
            pltpu.VMEM((REM,), jnp.int32),
            pltpu.VMEM((CH, D), jnp.float32),
            pltpu.VMEM_SHARED((N_PAD, D), jnp.float32),
            pltpu.SemaphoreType.DMA,
        ],
        name="sc_row_agg",
    )


BR = 400
GRID = N // BR


def _prep_body(deg0_ref, deg1_ref, x_ref, hs_ref, dis_ref):
    deg = deg0_ref[:, 0:1] + deg1_ref[:, 0:1] + 1.0
    dis = lax.rsqrt(deg)
    dis_ref[...] = dis
    hs_ref[...] = x_ref[...] * dis


def _tc_prep(deg0, deg1, x):
    return pl.pallas_call(
        _prep_body,
        grid=(GRID,),
        in_specs=[
            pl.BlockSpec((BR, DEGW), lambda i: (i, 0)),
            pl.BlockSpec((BR, DEGW), lambda i: (i, 0)),
            pl.BlockSpec((BR, D), lambda i: (i, 0)),
        ],
        out_specs=(
            pl.BlockSpec((BR, D), lambda i: (i, 0)),
            pl.BlockSpec((BR, 1), lambda i: (i, 0)),
        ),
        out_shape=(
            jax.ShapeDtypeStruct((N_PAD, D), jnp.float32),
            jax.ShapeDtypeStruct((N, 1), jnp.float32),
        ),
    )(deg0, deg1, x)


def _layer1_body(acc0_ref, acc1_ref, hs_ref, dis_ref, w1_ref, b1_ref,
                 out_ref):
    a = (acc0_ref[...] + acc1_ref[...] - hs_ref[...]) * dis_ref[...]
    h1 = jnp.dot(a, w1_ref[...], preferred_element_type=jnp.float32)
    h1 = jnp.maximum(h1 + b1_ref[...], 0.0)
    out_ref[...] = h1 * dis_ref[...]


def _tc_layer1(acc0, acc1, hs, dis, W1, b1r):
    return pl.pallas_call(
        _layer1_body,
        grid=(GRID,),
        in_specs=[
            pl.BlockSpec((BR, D), lambda i: (i, 0)),
            pl.BlockSpec((BR, D), lambda i: (i, 0)),
            pl.BlockSpec((BR, D), lambda i: (i, 0)),
            pl.BlockSpec((BR, 1), lambda i: (i, 0)),
            pl.BlockSpec((D, D), lambda i: (0, 0)),
            pl.BlockSpec((1, D), lambda i: (0, 0)),
        ],
        out_specs=pl.BlockSpec((BR, D), lambda i: (i, 0)),
        out_shape=jax.ShapeDtypeStruct((N_PAD, D), jnp.float32),
    )(acc0, acc1, hs, dis, W1, b1r)


def _layer2_body(acc0_ref, acc1_ref, h1s_ref, dis_ref, w2_ref, b2_ref,
                 wl_ref, bl_ref, out_ref):
    a = (acc0_ref[...] + acc1_ref[...] - h1s_ref[...]) * dis_ref[...]
    h2 = jnp.dot(a, w2_ref[...], preferred_element_type=jnp.float32)
    h2 = h2 + b2_ref[...]
    out = lax.dot_general(h2, wl_ref[...], (((1,), (1,)), ((), ())),
                          preferred_element_type=jnp.float32)
    out_ref[...] = out + bl_ref[...]


def _tc_layer2(acc0, acc1, h1s, dis, W2, b2r, Wl, blr):
    return pl.pallas_call(
        _layer2_body,
        grid=(GRID,),
        in_specs=[
            pl.BlockSpec((BR, D), lambda i: (i, 0)),
            pl.BlockSpec((BR, D), lambda i: (i, 0)),
            pl.BlockSpec((BR, D), lambda i: (i, 0)),
            pl.BlockSpec((BR, 1), lambda i: (i, 0)),
            pl.BlockSpec((D, NCLS), lambda i: (0, 0)),
            pl.BlockSpec((1, NCLS), lambda i: (0, 0)),
            pl.BlockSpec((NCLS, NCLS), lambda i: (0, 0)),
            pl.BlockSpec((1, NCLS), lambda i: (0, 0)),
        ],
        out_specs=pl.BlockSpec((BR, NCLS), lambda i: (i, 0)),
        out_shape=jax.ShapeDtypeStruct((N, NCLS), jnp.float32),
    )(acc0, acc1, h1s, dis, W2, b2r, Wl, blr)


def kernel(x, edge_index, W1, b1, W2, b2, Wl, bl):
    src = edge_index[0]
    dst = edge_index[1]

    deg0, deg1 = _sc_deg()(dst)
    hs, dis = _tc_prep(deg0, deg1, x)
    a10, a11 = _sc_agg()(hs, src, dst)
    h1s = _tc_layer1(a10, a11, hs, dis, W1, b1.reshape(1, D))

    a20, a21 = _sc_agg()(h1s, src, dst)
    return _tc_layer2(a20, a21, h1s, dis, W2, b2.reshape(1, NCLS),
                      Wl, bl.reshape(1, NCLS))


# speedup vs baseline: 26.9019x; 1.5583x over previous
import functools

import jax
import jax.numpy as jnp
from jax import lax
from jax.experimental import pallas as pl
from jax.experimental.pallas import tpu as pltpu
from jax.experimental.pallas import tpu_sc as plsc

N, E, D = 10000, 320000, 128
NCLS = 1250
NC, NS = 2, 16
NW = NC * NS
EPW = E // NW
CH = 128
NCHUNK = EPW // CH
REM = EPW - NCHUNK * CH
RPS = 640
N_PAD = NS * RPS
RQ = RPS // CH
DEGW = 128


def _mesh():
    return plsc.VectorSubcoreMesh(
        core_axis_name="c", subcore_axis_name="s",
        num_cores=NC, num_subcores=NS)


def _fill_wide(ref, n, val):
    vec = jnp.zeros((16,), jnp.float32) + val

    def body(i, _):
        for j in range(DEGW // 16):
            ref[i, pl.ds(j * 16, 16)] = vec
        return 0
    lax.fori_loop(0, n, body, 0)


def _mkidx(idx_v, base):
    def body(j, _):
        idx_v[pl.ds(j * 16, 16)] = lax.iota(jnp.int32, 16) + base + j * 16
        return 0
    lax.fori_loop(0, CH // 16, body, 0)


def _sc_deg_body(dst_hbm, out0_hbm, out1_hbm,
                 idx_v, idx2_v, idxr_v, ones_v, stage_v, hist_sh,
                 isem0, isem1):
    c = lax.axis_index("c")
    s = lax.axis_index("s")
    wid = c * NS + s
    base = s * RPS

    _fill_wide(stage_v, CH, 0.0)
    for q in range(RQ):
        _mkidx(idx_v, base + q * CH)
        pltpu.sync_copy(stage_v, hist_sh.at[idx_v])
    _fill_wide(ones_v, CH, 1.0)
    plsc.subcore_barrier()

    ebase = wid * EPW
    idx = (idx_v, idx2_v)
    isem = (isem0, isem1)

    def start_idx(b, i):
        pltpu.async_copy(dst_hbm.at[pl.ds(ebase + i * CH, CH)],
                         idx[b], isem[b])

    def wait_idx(b):
        pltpu.make_async_copy(dst_hbm.at[pl.ds(0, CH)], idx[b],
                              isem[b]).wait()

    start_idx(0, 0)
    start_idx(1, 1)

    def pair(p, _):
        for b in (0, 1):
            i = 2 * p + b
            wait_idx(b)
            pltpu.sync_copy(ones_v, hist_sh.at[idx[b]], add=True)

            @pl.when(i + 2 < NCHUNK)
            def _():
                start_idx(b, i + 2)
        return 0
    lax.fori_loop(0, NCHUNK // 2, pair, 0)

    pltpu.sync_copy(dst_hbm.at[pl.ds(ebase + NCHUNK * CH, REM)], idxr_v)
    pltpu.sync_copy(ones_v.at[pl.ds(0, REM)], hist_sh.at[idxr_v], add=True)

    plsc.subcore_barrier()

    def wout(out_hbm):
        for q in range(RQ):
            _mkidx(idx_v, base + q * CH)
            pltpu.sync_copy(hist_sh.at[idx_v], stage_v)
            pltpu.sync_copy(
                stage_v,
                out_hbm.at[pl.ds(pl.multiple_of(base + q * CH, 8), CH)])

    @pl.when(c == 0)
    def _():
        wout(out0_hbm)

    @pl.when(c == 1)
    def _():
        wout(out1_hbm)


@functools.cache
def _sc_deg():
    return pl.kernel(
        _sc_deg_body,
        out_type=(jax.ShapeDtypeStruct((N_PAD, DEGW), jnp.float32),
                  jax.ShapeDtypeStruct((N_PAD, DEGW), jnp.float32)),
        mesh=_mesh(),
        scratch_types=[
            pltpu.VMEM((CH,), jnp.int32),
            pltpu.VMEM((CH,), jnp.int32),
            pltpu.VMEM((REM,), jnp.int32),
            pltpu.VMEM((CH, DEGW), jnp.float32),
            pltpu.VMEM((CH, DEGW), jnp.float32),
            pltpu.VMEM_SHARED((N_PAD, DEGW), jnp.float32),
            pltpu.SemaphoreType.DMA,
            pltpu.SemaphoreType.DMA,
        ],
        name="sc_degree",
    )


def _sc_agg_body(hs_hbm, src_hbm, dst_hbm, out0_hbm, out1_hbm,
                 si_v, si2_v, di_v, di2_v, sir_v, dir_v, rows_v, rows2_v,
                 acc_sh, isem0, isem1, gsem0, gsem1):
    c = lax.axis_index("c")
    s = lax.axis_index("s")
    wid = c * NS + s
    base = s * RPS

    for q in range(RQ):
        _mkidx(si_v, base + q * CH)
        pltpu.sync_copy(
            hs_hbm.at[pl.ds(pl.multiple_of(base + q * CH, 8), CH)], rows_v)
        pltpu.sync_copy(rows_v, acc_sh.at[si_v])
    plsc.subcore_barrier()

    ebase = wid * EPW
    si = (si_v, si2_v)
    di = (di_v, di2_v)
    rows = (rows_v, rows2_v)
    isem = (isem0, isem1)
    gsem = (gsem0, gsem1)

    def start_idx(b, i):
        off = ebase + i * CH
        pltpu.async_copy(src_hbm.at[pl.ds(off, CH)], si[b], isem[b])
        pltpu.async_copy(dst_hbm.at[pl.ds(off, CH)], di[b], isem[b])

    def wait_idx(b):
        pltpu.make_async_copy(src_hbm.at[pl.ds(0, CH)], si[b], isem[b]).wait()
        pltpu.make_async_copy(dst_hbm.at[pl.ds(0, CH)], di[b], isem[b]).wait()

    def start_gather(b):
        pltpu.async_copy(hs_hbm.at[si[b]], rows[b], gsem[b])

    def wait_gather(b):
        pltpu.make_async_copy(hs_hbm.at[si[b]], rows[b], gsem[b]).wait()

    start_idx(0, 0)
    wait_idx(0)
    start_gather(0)
    start_idx(1, 1)

    def pair(p, _):
        for b in (0, 1):
            i = 2 * p + b
            nb = 1 - b
            @pl.when(i + 1 < NCHUNK)
            def _():
                wait_idx(nb)
                start_gather(nb)
            wait_gather(b)
            pltpu.sync_copy(rows[b], acc_sh.at[di[b]], add=True)

            @pl.when(i + 2 < NCHUNK)
            def _():
                start_idx(b, i + 2)
        return 0
    lax.fori_loop(0, NCHUNK // 2, pair, 0)

    off = ebase + NCHUNK * CH
    pltpu.sync_copy(src_hbm.at[pl.ds(off, REM)], sir_v)
    pltpu.sync_copy(dst_hbm.at[pl.ds(off, REM)], dir_v)
    pltpu.async_copy(hs_hbm.at[sir_v], rows_v.at[pl.ds(0, REM)], gsem0).wait()
    pltpu.sync_copy(rows_v.at[pl.ds(0, REM)], acc_sh.at[dir_v], add=True)

    plsc.subcore_barrier()

    def wout(out_hbm):
        for q in range(RQ):
            _mkidx(si_v, base + q * CH)
            pltpu.sync_copy(acc_sh.at[si_v], rows_v)
            pltpu.sync_copy(
                rows_v,
                out_hbm.at[pl.ds(pl.multiple_of(base + q * CH, 8), CH)])

    @pl.when(c == 0)
    def _():
        wout(out0_hbm)

    @pl.when(c == 1)
    def _():
        wout(out1_hbm)


@functools.cache
def _sc_agg():
    return pl.kernel(
        _sc_agg_body,
        out_type=(jax.ShapeDtypeStruct((N_PAD, D), jnp.float32),
                  jax.ShapeDtypeStruct((N_PAD, D), jnp.float32)),
        mesh=_mesh(),
        scratch_types=[
            pltpu.VMEM((CH,), jnp.int32),
            pltpu.VMEM((CH,), jnp.int32),
            pltpu.VMEM((CH,), jnp.int32),
            pltpu.VMEM((CH,), jnp.int32),
            pltpu.VMEM((REM,), jnp.int32),
            pltpu.VMEM((REM,), jnp.int32),
            pltpu.VMEM((CH, D), jnp.float32),
            pltpu.VMEM((CH, D), jnp.float32),
            pltpu.VMEM_SHARED((N_PAD, D), jnp.float32),
            pltpu.SemaphoreType.DMA,
            pltpu.SemaphoreType.DMA,
            pltpu.SemaphoreType.DMA,
            pltpu.SemaphoreType.DMA,
        ],
        name="sc_row_agg",
    )


BR = 400
GRID = N // BR


def _prep_body(deg0_ref, deg1_ref, x_ref, hs_ref, dis_ref):
    deg = deg0_ref[:, 0:1] + deg1_ref[:, 0:1] + 1.0
    dis = lax.rsqrt(deg)
    dis_ref[...] = dis
    hs_ref[...] = x_ref[...] * dis


def _tc_prep(deg0, deg1, x):
    return pl.pallas_call(
        _prep_body,
        grid=(GRID,),
        in_specs=[
            pl.BlockSpec((BR, DEGW), lambda i: (i, 0)),
            pl.BlockSpec((BR, DEGW), lambda i: (i, 0)),
            pl.BlockSpec((BR, D), lambda i: (i, 0)),
        ],
        out_specs=(
            pl.BlockSpec((BR, D), lambda i: (i, 0)),
            pl.BlockSpec((BR, 1), lambda i: (i, 0)),
        ),
        out_shape=(
            jax.ShapeDtypeStruct((N_PAD, D), jnp.float32),
            jax.ShapeDtypeStruct((N, 1), jnp.float32),
        ),
    )(deg0, deg1, x)


def _layer1_body(acc0_ref, acc1_ref, hs_ref, dis_ref, w1_ref, b1_ref,
                 out_ref):
    a = (acc0_ref[...] + acc1_ref[...] - hs_ref[...]) * dis_ref[...]
    h1 = jnp.dot(a, w1_ref[...], preferred_element_type=jnp.float32)
    h1 = jnp.maximum(h1 + b1_ref[...], 0.0)
    out_ref[...] = h1 * dis_ref[...]


def _tc_layer1(acc0, acc1, hs, dis, W1, b1r):
    return pl.pallas_call(
        _layer1_body,
        grid=(GRID,),
        in_specs=[
            pl.BlockSpec((BR, D), lambda i: (i, 0)),
            pl.BlockSpec((BR, D), lambda i: (i, 0)),
            pl.BlockSpec((BR, D), lambda i: (i, 0)),
            pl.BlockSpec((BR, 1), lambda i: (i, 0)),
            pl.BlockSpec((D, D), lambda i: (0, 0)),
            pl.BlockSpec((1, D), lambda i: (0, 0)),
        ],
        out_specs=pl.BlockSpec((BR, D), lambda i: (i, 0)),
        out_shape=jax.ShapeDtypeStruct((N_PAD, D), jnp.float32),
    )(acc0, acc1, hs, dis, W1, b1r)


def _layer2_body(acc0_ref, acc1_ref, h1s_ref, dis_ref, w2_ref, b2_ref,
                 wl_ref, bl_ref, out_ref):
    a = (acc0_ref[...] + acc1_ref[...] - h1s_ref[...]) * dis_ref[...]
    h2 = jnp.dot(a, w2_ref[...], preferred_element_type=jnp.float32)
    h2 = h2 + b2_ref[...]
    out = lax.dot_general(h2, wl_ref[...], (((1,), (1,)), ((), ())),
                          preferred_element_type=jnp.float32)
    out_ref[...] = out + bl_ref[...]


def _tc_layer2(acc0, acc1, h1s, dis, W2, b2r, Wl, blr):
    return pl.pallas_call(
        _layer2_body,
        grid=(GRID,),
        in_specs=[
            pl.BlockSpec((BR, D), lambda i: (i, 0)),
            pl.BlockSpec((BR, D), lambda i: (i, 0)),
            pl.BlockSpec((BR, D), lambda i: (i, 0)),
            pl.BlockSpec((BR, 1), lambda i: (i, 0)),
            pl.BlockSpec((D, NCLS), lambda i: (0, 0)),
            pl.BlockSpec((1, NCLS), lambda i: (0, 0)),
            pl.BlockSpec((NCLS, NCLS), lambda i: (0, 0)),
            pl.BlockSpec((1, NCLS), lambda i: (0, 0)),
        ],
        out_specs=pl.BlockSpec((BR, NCLS), lambda i: (i, 0)),
        out_shape=jax.ShapeDtypeStruct((N, NCLS), jnp.float32),
    )(acc0, acc1, h1s, dis, W2, b2r, Wl, blr)


def kernel(x, edge_index, W1, b1, W2, b2, Wl, bl):
    src = edge_index[0]
    dst = edge_index[1]

    deg0, deg1 = _sc_deg()(dst)
    hs, dis = _tc_prep(deg0, deg1, x)
    a10, a11 = _sc_agg()(hs, src, dst)
    h1s = _tc_layer1(a10, a11, hs, dis, W1, b1.reshape(1, D))

    a20, a21 = _sc_agg()(h1s, src, dst)
    return _tc_layer2(a20, a21, h1s, dis, W2, b2.reshape(1, NCLS),
                      Wl, bl.reshape(1, NCLS))


# speedup vs baseline: 28.7210x; 1.0676x over previous
import functools

import jax
import jax.numpy as jnp
from jax import lax
from jax.experimental import pallas as pl
from jax.experimental.pallas import tpu as pltpu
from jax.experimental.pallas import tpu_sc as plsc

N, E, D = 10000, 320000, 128
NCLS = 1250
NC, NS = 2, 16
NW = NC * NS
EPW = E // NW
CH = 128
NCHUNK = EPW // CH
REM = EPW - NCHUNK * CH
RPS = 640
N_PAD = NS * RPS
RQ = RPS // CH
DEGW = 128


def _mesh():
    return plsc.VectorSubcoreMesh(
        core_axis_name="c", subcore_axis_name="s",
        num_cores=NC, num_subcores=NS)


def _fill_wide(ref, n, val):
    vec = jnp.zeros((16,), jnp.float32) + val

    def body(i, _):
        for j in range(DEGW // 16):
            ref[i, pl.ds(j * 16, 16)] = vec
        return 0
    lax.fori_loop(0, n, body, 0)


def _mkidx(idx_v, base):
    def body(j, _):
        idx_v[pl.ds(j * 16, 16)] = lax.iota(jnp.int32, 16) + base + j * 16
        return 0
    lax.fori_loop(0, CH // 16, body, 0)


def _sc_deg_body(dst_hbm, out0_hbm, out1_hbm,
                 idx_v, idx2_v, idx3_v, idx4_v, idxr_v, ones_v, stage_v,
                 hist_sh, isem0, isem1, isem2, isem3, ssem):
    c = lax.axis_index("c")
    s = lax.axis_index("s")
    wid = c * NS + s
    base = s * RPS

    _fill_wide(stage_v, CH, 0.0)
    for q in range(RQ):
        _mkidx(idx_v, base + q * CH)
        pltpu.sync_copy(stage_v, hist_sh.at[idx_v])
    _fill_wide(ones_v, CH, 1.0)
    plsc.subcore_barrier()

    ebase = wid * EPW
    idx = (idx_v, idx2_v, idx3_v, idx4_v)
    isem = (isem0, isem1, isem2, isem3)

    def start_idx(q, i):
        pltpu.async_copy(dst_hbm.at[pl.ds(ebase + i * CH, CH)],
                         idx[q], isem[q])

    def wait_idx(q):
        pltpu.make_async_copy(dst_hbm.at[pl.ds(0, CH)], idx[q],
                              isem[q]).wait()

    def drain_scatter():
        pltpu.make_async_copy(ones_v, hist_sh.at[idx[0]], ssem).wait()

    for q in range(3):
        start_idx(q, q)

    def quad(t, _):
        for u in range(4):
            i4 = 4 * t + u
            wait_idx(u)
            pltpu.async_copy(ones_v, hist_sh.at[idx[u]], ssem, add=True)

            @pl.when(i4 >= 1)
            def _():
                drain_scatter()

            @pl.when(i4 + 3 < NCHUNK)
            def _():
                start_idx((u + 3) % 4, i4 + 3)
        return 0
    lax.fori_loop(0, NCHUNK // 4, quad, 0)

    for i4 in (NCHUNK - 2, NCHUNK - 1):
        q = i4 % 4
        wait_idx(q)
        pltpu.async_copy(ones_v, hist_sh.at[idx[q]], ssem, add=True)
        drain_scatter()
    drain_scatter()

    pltpu.sync_copy(dst_hbm.at[pl.ds(ebase + NCHUNK * CH, REM)], idxr_v)
    pltpu.sync_copy(ones_v.at[pl.ds(0, REM)], hist_sh.at[idxr_v], add=True)

    plsc.subcore_barrier()

    def wout(out_hbm):
        for q in range(RQ):
            _mkidx(idx_v, base + q * CH)
            pltpu.sync_copy(hist_sh.at[idx_v], stage_v)
            pltpu.sync_copy(
                stage_v,
                out_hbm.at[pl.ds(pl.multiple_of(base + q * CH, 8), CH)])

    @pl.when(c == 0)
    def _():
        wout(out0_hbm)

    @pl.when(c == 1)
    def _():
        wout(out1_hbm)


@functools.cache
def _sc_deg():
    return pl.kernel(
        _sc_deg_body,
        out_type=(jax.ShapeDtypeStruct((N_PAD, DEGW), jnp.float32),
                  jax.ShapeDtypeStruct((N_PAD, DEGW), jnp.float32)),
        mesh=_mesh(),
        scratch_types=[
            pltpu.VMEM((CH,), jnp.int32),
            pltpu.VMEM((CH,), jnp.int32),
            pltpu.VMEM((CH,), jnp.int32),
            pltpu.VMEM((CH,), jnp.int32),
            pltpu.VMEM((REM,), jnp.int32),
            pltpu.VMEM((CH, DEGW), jnp.float32),
            pltpu.VMEM((CH, DEGW), jnp.float32),
            pltpu.VMEM_SHARED((N_PAD, DEGW), jnp.float32),
            pltpu.SemaphoreType.DMA,
            pltpu.SemaphoreType.DMA,
            pltpu.SemaphoreType.DMA,
            pltpu.SemaphoreType.DMA,
            pltpu.SemaphoreType.DMA,
        ],
        name="sc_degree",
    )


def _sc_agg_body(hs_hbm, src_hbm, dst_hbm, out0_hbm, out1_hbm,
                 si_v, si2_v, si3_v, si4_v, di_v, di2_v, di3_v, di4_v,
                 sir_v, dir_v, rows_v, rows2_v, acc_sh,
                 isem0, isem1, isem2, isem3, gsem0, gsem1, ssem0, ssem1):
    c = lax.axis_index("c")
    s = lax.axis_index("s")
    wid = c * NS + s
    base = s * RPS

    for q in range(RQ):
        _mkidx(si_v, base + q * CH)
        pltpu.sync_copy(
            hs_hbm.at[pl.ds(pl.multiple_of(base + q * CH, 8), CH)], rows_v)
        pltpu.sync_copy(rows_v, acc_sh.at[si_v])
    plsc.subcore_barrier()

    ebase = wid * EPW
    si = (si_v, si2_v, si3_v, si4_v)
    di = (di_v, di2_v, di3_v, di4_v)
    rows = (rows_v, rows2_v)
    isem = (isem0, isem1, isem2, isem3)
    gsem = (gsem0, gsem1)
    ssem = (ssem0, ssem1)

    def start_idx(q, i):
        off = ebase + i * CH
        pltpu.async_copy(src_hbm.at[pl.ds(off, CH)], si[q], isem[q])
        pltpu.async_copy(dst_hbm.at[pl.ds(off, CH)], di[q], isem[q])

    def wait_idx(q):
        pltpu.make_async_copy(src_hbm.at[pl.ds(0, CH)], si[q], isem[q]).wait()
        pltpu.make_async_copy(dst_hbm.at[pl.ds(0, CH)], di[q], isem[q]).wait()

    def start_gather(b, q):
        pltpu.async_copy(hs_hbm.at[si[q]], rows[b], gsem[b])

    def wait_gather(b, q):
        pltpu.make_async_copy(hs_hbm.at[si[q]], rows[b], gsem[b]).wait()

    def start_scatter(b, q):
        pltpu.async_copy(rows[b], acc_sh.at[di[q]], ssem[b], add=True)

    def drain_scatter(b):
        pltpu.make_async_copy(rows[b], acc_sh.at[di[0]], ssem[b]).wait()

    for q in range(3):
        start_idx(q, q)
    wait_idx(0)
    start_gather(0, 0)

    def step(i, b, q):
        nb, nq = 1 - b, (q + 1) % 4
        wait_idx(nq)
        pl.when(i >= 1)(lambda: drain_scatter(nb))
        start_gather(nb, nq)
        wait_gather(b, q)
        start_scatter(b, q)
        pl.when(i + 3 < NCHUNK)(lambda: start_idx((q + 3) % 4, i + 3))

    def quad(t, _):
        for u in range(4):
            step(4 * t + u, u % 2, u)
        return 0
    lax.fori_loop(0, NCHUNK // 4, quad, 0)

    i = NCHUNK - 2
    wait_idx((i + 1) % 4)
    drain_scatter(1)
    start_gather(1, (i + 1) % 4)
    wait_gather(0, i % 4)
    start_scatter(0, i % 4)
    i = NCHUNK - 1
    drain_scatter(0)
    wait_gather(1, i % 4)
    start_scatter(1, i % 4)
    drain_scatter(1)

    off = ebase + NCHUNK * CH
    pltpu.sync_copy(src_hbm.at[pl.ds(off, REM)], sir_v)
    pltpu.sync_copy(dst_hbm.at[pl.ds(off, REM)], dir_v)
    pltpu.async_copy(hs_hbm.at[sir_v], rows_v.at[pl.ds(0, REM)], gsem0).wait()
    pltpu.sync_copy(rows_v.at[pl.ds(0, REM)], acc_sh.at[dir_v], add=True)

    plsc.subcore_barrier()

    def wout(out_hbm):
        for q in range(RQ):
            _mkidx(si_v, base + q * CH)
            pltpu.sync_copy(acc_sh.at[si_v], rows_v)
            pltpu.sync_copy(
                rows_v,
                out_hbm.at[pl.ds(pl.multiple_of(base + q * CH, 8), CH)])

    @pl.when(c == 0)
    def _():
        wout(out0_hbm)

    @pl.when(c == 1)
    def _():
        wout(out1_hbm)


@functools.cache
def _sc_agg():
    return pl.kernel(
        _sc_agg_body,
        out_type=(jax.ShapeDtypeStruct((N_PAD, D), jnp.float32),
                  jax.ShapeDtypeStruct((N_PAD, D), jnp.float32)),
        mesh=_mesh(),
        scratch_types=(
            [pltpu.VMEM((CH,), jnp.int32)] * 8
            + [pltpu.VMEM((REM,), jnp.int32)] * 2
            + [pltpu.VMEM((CH, D), jnp.float32)] * 2
            + [pltpu.VMEM_SHARED((N_PAD, D), jnp.float32)]
            + [pltpu.SemaphoreType.DMA] * 8
        ),
        name="sc_row_agg",
    )


BR = 400
GRID = N // BR


def _prep_body(deg0_ref, deg1_ref, x_ref, hs_ref, dis_ref):
    deg = deg0_ref[:, 0:1] + deg1_ref[:, 0:1] + 1.0
    dis = lax.rsqrt(deg)
    dis_ref[...] = dis
    hs_ref[...] = x_ref[...] * dis


def _tc_prep(deg0, deg1, x):
    return pl.pallas_call(
        _prep_body,
        grid=(GRID,),
        in_specs=[
            pl.BlockSpec((BR, DEGW), lambda i: (i, 0)),
            pl.BlockSpec((BR, DEGW), lambda i: (i, 0)),
            pl.BlockSpec((BR, D), lambda i: (i, 0)),
        ],
        out_specs=(
            pl.BlockSpec((BR, D), lambda i: (i, 0)),
            pl.BlockSpec((BR, 1), lambda i: (i, 0)),
        ),
        out_shape=(
            jax.ShapeDtypeStruct((N_PAD, D), jnp.float32),
            jax.ShapeDtypeStruct((N, 1), jnp.float32),
        ),
    )(deg0, deg1, x)


def _layer1_body(acc0_ref, acc1_ref, hs_ref, dis_ref, w1_ref, b1_ref,
                 out_ref):
    a = (acc0_ref[...] + acc1_ref[...] - hs_ref[...]) * dis_ref[...]
    h1 = jnp.dot(a, w1_ref[...], preferred_element_type=jnp.float32)
    h1 = jnp.maximum(h1 + b1_ref[...], 0.0)
    out_ref[...] = h1 * dis_ref[...]


def _tc_layer1(acc0, acc1, hs, dis, W1, b1r):
    return pl.pallas_call(
        _layer1_body,
        grid=(GRID,),
        in_specs=[
            pl.BlockSpec((BR, D), lambda i: (i, 0)),
            pl.BlockSpec((BR, D), lambda i: (i, 0)),
            pl.BlockSpec((BR, D), lambda i: (i, 0)),
            pl.BlockSpec((BR, 1), lambda i: (i, 0)),
            pl.BlockSpec((D, D), lambda i: (0, 0)),
            pl.BlockSpec((1, D), lambda i: (0, 0)),
        ],
        out_specs=pl.BlockSpec((BR, D), lambda i: (i, 0)),
        out_shape=jax.ShapeDtypeStruct((N_PAD, D), jnp.float32),
    )(acc0, acc1, hs, dis, W1, b1r)


def _layer2_body(acc0_ref, acc1_ref, h1s_ref, dis_ref, w2_ref, b2_ref,
                 wl_ref, bl_ref, out_ref):
    a = (acc0_ref[...] + acc1_ref[...] - h1s_ref[...]) * dis_ref[...]
    h2 = jnp.dot(a, w2_ref[...], preferred_element_type=jnp.float32)
    h2 = h2 + b2_ref[...]
    out = lax.dot_general(h2, wl_ref[...], (((1,), (1,)), ((), ())),
                          preferred_element_type=jnp.float32)
    out_ref[...] = out + bl_ref[...]


def _tc_layer2(acc0, acc1, h1s, dis, W2, b2r, Wl, blr):
    return pl.pallas_call(
        _layer2_body,
        grid=(GRID,),
        in_specs=[
            pl.BlockSpec((BR, D), lambda i: (i, 0)),
            pl.BlockSpec((BR, D), lambda i: (i, 0)),
            pl.BlockSpec((BR, D), lambda i: (i, 0)),
            pl.BlockSpec((BR, 1), lambda i: (i, 0)),
            pl.BlockSpec((D, NCLS), lambda i: (0, 0)),
            pl.BlockSpec((1, NCLS), lambda i: (0, 0)),
            pl.BlockSpec((NCLS, NCLS), lambda i: (0, 0)),
            pl.BlockSpec((1, NCLS), lambda i: (0, 0)),
        ],
        out_specs=pl.BlockSpec((BR, NCLS), lambda i: (i, 0)),
        out_shape=jax.ShapeDtypeStruct((N, NCLS), jnp.float32),
    )(acc0, acc1, h1s, dis, W2, b2r, Wl, blr)


def kernel(x, edge_index, W1, b1, W2, b2, Wl, bl):
    src = edge_index[0]
    dst = edge_index[1]

    deg0, deg1 = _sc_deg()(dst)
    hs, dis = _tc_prep(deg0, deg1, x)
    a10, a11 = _sc_agg()(hs, src, dst)
    h1s = _tc_layer1(a10, a11, hs, dis, W1, b1.reshape(1, D))

    a20, a21 = _sc_agg()(h1s, src, dst)
    return _tc_layer2(a20, a21, h1s, dis, W2, b2.reshape(1, NCLS),
                      Wl, bl.reshape(1, NCLS))
